# Initial kernel scaffold; baseline (speedup 1.0000x reference)
import jax
import jax.numpy as jnp
from jax.experimental import pallas as pl


def kernel(x, additional, W_lin, bias, ffm_tables):
    raise NotImplementedError("write your pallas kernel here")


# speedup vs baseline: 8.5457x; 8.5457x over previous
import functools

import jax
import jax.numpy as jnp
import numpy as np
from jax import lax
from jax.experimental import pallas as pl
from jax.experimental.pallas import tpu as pltpu
from jax.experimental.pallas import tpu_sc as plsc

NUM_FIELDS_RAW = 39
FIELD_DIM = 2560
EMBED_DIM = 16
BATCH = 4096

F = 30
FPAD = 32
TOTAL = F * FIELD_DIM
NC, NS, L = 2, 16, 16
NW = NC * NS
BPW = BATCH // NW
NCHUNK = 8


def _field_offsets_i32():
    sel = np.full(NUM_FIELDS_RAW, FIELD_DIM, dtype=np.int64)
    sel = np.hstack((sel[:3], sel[4:8], sel[10:15], sel[17:19], sel[21:24], sel[26:]))
    return np.array((0, *np.cumsum(sel)[:-1]), dtype=np.int32)


def _select_cols(x):
    return jnp.concatenate(
        (x[:, :3], x[:, 4:8], x[:, 10:15], x[:, 17:19], x[:, 21:24], x[:, 26:]),
        axis=1)


def _ffm_kernel(tab_hbm, wpad_hbm, xi_hbm, bias_hbm, out_hbm,
                xi_v, idx_v, widx_v, rows_v, wrows_v, z_v, bias_v, sem):
    wid = lax.axis_index("s") * NC + lax.axis_index("c")
    base = wid * BPW
    pltpu.sync_copy(xi_hbm.at[pl.ds(base, BPW)], xi_v)
    pltpu.sync_copy(bias_hbm, bias_v)

    lanes = lax.iota(jnp.int32, L)

    zero_i = jnp.zeros((L,), jnp.int32)
    for o in range(F * FPAD - 7 * 128, 128, L):
        idx_v[NCHUNK - 1, pl.ds(o, L)] = zero_i

    def sample_body(s, zvec):
        xa = xi_v[s, pl.ds(0, L)]
        xb = xi_v[s, pl.ds(L, L)]
        widx_v[pl.ds(0, L)] = xa
        widx_v[pl.ds(L, L)] = xb
        for t in range(F):
            flat = t * FPAD
            c, o = divmod(flat, 128)
            idx_v[c, pl.ds(o, L)] = xa + (t * TOTAL)
            idx_v[c, pl.ds(o + L, L)] = xb + (t * TOTAL)
        handles = [
            pltpu.async_copy(tab_hbm.at[idx_v.at[c]], rows_v.at[c], sem)
            for c in range(NCHUNK)
        ]
        hw = pltpu.async_copy(wpad_hbm.at[widx_v], wrows_v, sem)
        for h in handles:
            h.wait()
        hw.wait()

        acc = jnp.zeros((L,), jnp.float32)
        for f in range(F):
            acc = acc + wrows_v[f]
        for i in range(F):
            for j in range(i + 1, F):
                si = i * FPAD + j
                sj = j * FPAD + i
                acc = acc + rows_v[si // 128, si % 128] * rows_v[sj // 128, sj % 128]
        for sh in (1, 2, 4, 8):
            acc = acc + jnp.take_along_axis(
                acc, lanes ^ sh, axis=0, mode="promise_in_bounds")
        lane = s % L
        zvec = jnp.where(lanes == lane, acc, zvec)

        @pl.when(lane == L - 1)
        def _():
            z_v[pl.ds(pl.multiple_of((s // L) * L, L), L)] = zvec

        return zvec

    lax.fori_loop(0, BPW, sample_body, jnp.zeros((L,), jnp.float32))

    for g in range(BPW // L):
        zz = z_v[pl.ds(g * L, L)]
        z_v[pl.ds(g * L, L)] = 1.0 / (1.0 + jnp.exp(-(zz + bias_v[...])))
    pltpu.sync_copy(z_v, out_hbm.at[pl.ds(base, BPW)])


@jax.jit
def _run(tab_flat, wpad, xi_pad, bias16):
    mesh = plsc.VectorSubcoreMesh(
        core_axis_name="c", subcore_axis_name="s", num_cores=NC, num_subcores=NS)
    return pl.kernel(
        _ffm_kernel,
        out_type=jax.ShapeDtypeStruct((BATCH,), jnp.float32),
        mesh=mesh,
        compiler_params=pltpu.CompilerParams(use_tc_tiling_on_sc=False),
        scratch_types=[
            pltpu.VMEM((BPW, FPAD), jnp.int32),
            pltpu.VMEM((NCHUNK, 128), jnp.int32),
            pltpu.VMEM((FPAD,), jnp.int32),
            pltpu.VMEM((NCHUNK, 128, L), jnp.float32),
            pltpu.VMEM((FPAD, L), jnp.float32),
            pltpu.VMEM((BPW,), jnp.float32),
            pltpu.VMEM((L,), jnp.float32),
            pltpu.SemaphoreType.DMA,
        ],
    )(tab_flat, wpad, xi_pad, bias16)


def kernel(x, additional, W_lin, bias, ffm_tables):
    offsets = jnp.asarray(_field_offsets_i32())
    xi = _select_cols(x).astype(jnp.int32) + offsets[None, :]
    xi_pad = jnp.pad(xi, ((0, 0), (0, FPAD - F)))
    tab_flat = ffm_tables.reshape(F * TOTAL, EMBED_DIM)
    wpad = jnp.pad(W_lin.astype(jnp.float32), ((0, 0), (0, L - 1)))
    bias16 = jnp.broadcast_to(bias.astype(jnp.float32), (L,))
    return _run(tab_flat, wpad, xi_pad, bias16)


# speedup vs baseline: 31.2045x; 3.6515x over previous
import jax
import jax.numpy as jnp
import numpy as np
from jax import lax
from jax.experimental import pallas as pl
from jax.experimental.pallas import tpu as pltpu
from jax.experimental.pallas import tpu_sc as plsc

NUM_FIELDS_RAW = 39
FIELD_DIM = 2560
EMBED_DIM = 16
BATCH = 4096

F = 30
FPAD = 32
TOTAL = F * FIELD_DIM
ROWLEN = F * EMBED_DIM
NC, NS, L = 2, 16, 16
NW = NC * NS
BPW = BATCH // NW


def _field_offsets_i32():
    sel = np.full(NUM_FIELDS_RAW, FIELD_DIM, dtype=np.int64)
    sel = np.hstack((sel[:3], sel[4:8], sel[10:15], sel[17:19], sel[21:24], sel[26:]))
    return np.array((0, *np.cumsum(sel)[:-1]), dtype=np.int32)


def _select_cols(x):
    return jnp.concatenate(
        (x[:, :3], x[:, 4:8], x[:, 10:15], x[:, 17:19], x[:, 21:24], x[:, 26:]),
        axis=1)


def _ffm_kernel(big_hbm, xi_hbm, bias_hbm, out_hbm,
                xi_v, idx_v, rows_v, z_v, bias_v, sem):
    wid = lax.axis_index("s") * NC + lax.axis_index("c")
    base = wid * BPW
    pltpu.sync_copy(xi_hbm.at[pl.ds(base, BPW)], xi_v)
    pltpu.sync_copy(bias_hbm, bias_v)

    lanes = lax.iota(jnp.int32, L)
    pa = jnp.where(lanes < 2, lanes + 14, 0)
    pb = jnp.where(lanes < 2, 0, lanes - 2)

    def sample_body(s, zvec):
        xa = xi_v[s, pl.ds(0, L)]
        xb = xi_v[s, pl.ds(L, L)]
        idx_v[pl.ds(0, L)] = xa
        tail = jnp.where(
            lanes < 2,
            jnp.take_along_axis(xa, pa, axis=0, mode="promise_in_bounds"),
            jnp.take_along_axis(xb, pb, axis=0, mode="promise_in_bounds"))
        idx_v[pl.ds(F - L, L)] = tail
        pltpu.async_copy(big_hbm.at[idx_v], rows_v, sem).wait()

        acc = jnp.zeros((L,), jnp.float32)
        for f in range(F):
            acc = acc + rows_v[f, pl.ds(EMBED_DIM * f, L)]
        for i in range(F):
            for j in range(i + 1, F):
                acc = acc + (rows_v[j, pl.ds(EMBED_DIM * i, L)]
                             * rows_v[i, pl.ds(EMBED_DIM * j, L)])
        for sh in (1, 2, 4, 8):
            acc = acc + jnp.take_along_axis(
                acc, lanes ^ sh, axis=0, mode="promise_in_bounds")
        lane = s % L
        zvec = jnp.where(lanes == lane, acc, zvec)

        @pl.when(lane == L - 1)
        def _():
            z_v[pl.ds(pl.multiple_of((s // L) * L, L), L)] = zvec

        return zvec

    lax.fori_loop(0, BPW, sample_body, jnp.zeros((L,), jnp.float32))

    for g in range(BPW // L):
        zz = z_v[pl.ds(g * L, L)]
        z_v[pl.ds(g * L, L)] = 1.0 / (1.0 + jnp.exp(-(zz + bias_v[...])))
    pltpu.sync_copy(z_v, out_hbm.at[pl.ds(base, BPW)])


@jax.jit
def _run(big, xi_pad, bias16):
    mesh = plsc.VectorSubcoreMesh(
        core_axis_name="c", subcore_axis_name="s", num_cores=NC, num_subcores=NS)
    return pl.kernel(
        _ffm_kernel,
        out_type=jax.ShapeDtypeStruct((BATCH,), jnp.float32),
        mesh=mesh,
        compiler_params=pltpu.CompilerParams(use_tc_tiling_on_sc=False),
        scratch_types=[
            pltpu.VMEM((BPW, FPAD), jnp.int32),
            pltpu.VMEM((F,), jnp.int32),
            pltpu.VMEM((F, ROWLEN), jnp.float32),
            pltpu.VMEM((BPW,), jnp.float32),
            pltpu.VMEM((L,), jnp.float32),
            pltpu.SemaphoreType.DMA,
        ],
    )(big, xi_pad, bias16)


def kernel(x, additional, W_lin, bias, ffm_tables):
    offsets = jnp.asarray(_field_offsets_i32())
    xi = _select_cols(x).astype(jnp.int32) + offsets[None, :]
    xi_pad = jnp.pad(xi, ((0, 0), (0, FPAD - F)))
    tabT = jnp.swapaxes(ffm_tables, 0, 1)
    rf = jnp.repeat(jnp.arange(F, dtype=jnp.int32), FIELD_DIM)
    m_diag = rf[:, None, None] == jnp.arange(F, dtype=jnp.int32)[None, :, None]
    lane0 = (jnp.arange(EMBED_DIM) == 0)[None, None, :]
    wb = W_lin.astype(jnp.float32)[:, :, None]
    big3 = jnp.where(m_diag & lane0, wb, jnp.where(m_diag, 0.0, tabT))
    big = big3.reshape(TOTAL, ROWLEN)
    bias16 = jnp.broadcast_to(bias.astype(jnp.float32), (L,))
    return _run(big, xi_pad, bias16)
